# Initial kernel scaffold; baseline (speedup 1.0000x reference)
import jax
import jax.numpy as jnp
from jax.experimental import pallas as pl


def kernel(routing_info, weights):
    raise NotImplementedError("write your pallas kernel here")


# speedup vs baseline: 1.2227x; 1.2227x over previous
import functools

import jax
import jax.numpy as jnp
from jax import lax
from jax.experimental import pallas as pl
from jax.experimental.pallas import tpu as pltpu
from jax.experimental.pallas import tpu_sc as plsc

_EPS = 1e-09
_N_TASKS = 1000
_N_SKILLS = 64
_N_SPLITS = 8
_BS = 16384
_D = _N_SKILLS * _N_SPLITS

_NC = 2
_NS = 16
_NW = _NC * _NS
_B_PER_W = _BS // _NW
_CH = 64
_N_CHUNKS = _B_PER_W // _CH


def _sc_body(idx_hbm, table_hbm, out_hbm, idx_v, buf_v, sem):
    wid = lax.axis_index("s") * _NC + lax.axis_index("c")
    base_row = wid * _B_PER_W
    pltpu.sync_copy(idx_hbm.at[pl.ds(base_row, _B_PER_W)], idx_v)

    def do_chunk(c, carry):
        row0 = base_row + c * _CH
        pltpu.async_copy(
            table_hbm.at[idx_v.at[pl.ds(c * _CH, _CH)]], buf_v, sem
        ).wait()

        lanes = lax.iota(jnp.int32, 16)
        _dnums = lax.GatherDimensionNumbers(
            offset_dims=(), collapsed_slice_dims=(0,), start_index_map=(0,)
        )

        def _lane_perm(v, idx):
            return lax.gather(
                v,
                idx.reshape(16, 1),
                _dnums,
                slice_sizes=(1,),
                mode=lax.GatherScatterMode.PROMISE_IN_BOUNDS,
            )

        def do_row(r, carry2):
            for g in range(_N_SPLITS):
                base = g * _N_SKILLS
                vals = []
                for j in range(_N_SKILLS // 16):
                    x = buf_v[r, pl.ds(base + j * 16, 16)]
                    vals.append(1.0 / (1.0 + jnp.exp(-x)))
                tot = (vals[0] + vals[1]) + (vals[2] + vals[3])
                for k in (8, 4, 2, 1):
                    tot = tot + _lane_perm(tot, lanes ^ k)
                inv = 1.0 / (tot + _EPS)
                for j in range(_N_SKILLS // 16):
                    buf_v[r, pl.ds(base + j * 16, 16)] = vals[j] * inv
            return carry2

        lax.fori_loop(0, _CH, do_row, 0)
        pltpu.sync_copy(buf_v, out_hbm.at[pl.ds(row0, _CH)])
        return carry

    lax.fori_loop(0, _N_CHUNKS, do_chunk, 0)


@functools.partial(
    pl.kernel,
    mesh=plsc.VectorSubcoreMesh(core_axis_name="c", subcore_axis_name="s"),
    out_type=jax.ShapeDtypeStruct((_BS, _D), jnp.float32),
    scratch_types=[
        pltpu.VMEM((_B_PER_W,), jnp.int32),
        pltpu.VMEM((_CH, _D), jnp.float32),
        pltpu.SemaphoreType.DMA,
    ],
)
def _poly_selector(idx_hbm, table_hbm, out_hbm, idx_v, buf_v, sem):
    _sc_body(idx_hbm, table_hbm, out_hbm, idx_v, buf_v, sem)


def kernel(routing_info, weights):
    idx = routing_info.reshape(-1).astype(jnp.int32)
    out = _poly_selector(idx, weights)
    return out.reshape(_BS, _N_SPLITS, _N_SKILLS)


# speedup vs baseline: 1.5646x; 1.2797x over previous
import functools

import jax
import jax.numpy as jnp
from jax import lax
from jax.experimental import pallas as pl
from jax.experimental.pallas import tpu as pltpu
from jax.experimental.pallas import tpu_sc as plsc

_EPS = 1e-09
_N_TASKS = 1000
_N_TASKS_PAD = 1024
_N_SKILLS = 64
_N_SPLITS = 8
_BS = 16384
_D = _N_SKILLS * _N_SPLITS

_NC = 2
_NS = 16
_NW = _NC * _NS
_B_PER_W = _BS // _NW
_T_PER_S = _N_TASKS_PAD // _NS
_CH = 64
_N_CHUNKS = _B_PER_W // _CH


def _normalize_rows(buf_v, n_rows):
    lanes = lax.iota(jnp.int32, 16)
    dnums = lax.GatherDimensionNumbers(
        offset_dims=(), collapsed_slice_dims=(0,), start_index_map=(0,)
    )

    def lane_perm(v, idx):
        return lax.gather(
            v,
            idx.reshape(16, 1),
            dnums,
            slice_sizes=(1,),
            mode=lax.GatherScatterMode.PROMISE_IN_BOUNDS,
        )

    def do_row(r, carry):
        for g in range(_N_SPLITS):
            base = g * _N_SKILLS
            vals = []
            for j in range(_N_SKILLS // 16):
                x = buf_v[r, pl.ds(base + j * 16, 16)]
                vals.append(1.0 / (1.0 + jnp.exp(-x)))
            tot = (vals[0] + vals[1]) + (vals[2] + vals[3])
            for k in (8, 4, 2, 1):
                tot = tot + lane_perm(tot, lanes ^ k)
            inv = 1.0 / (tot + _EPS)
            for j in range(_N_SKILLS // 16):
                buf_v[r, pl.ds(base + j * 16, 16)] = vals[j] * inv
        return carry

    lax.fori_loop(0, n_rows, do_row, 0)


def _sc_body(idx_hbm, table_hbm, out_hbm, ptable_hbm, idx_v, buf_a, buf_b,
             gsem, osem_a, osem_b):
    sid = lax.axis_index("s")
    cid = lax.axis_index("c")
    wid = sid * _NC + cid

    trow0 = sid * _T_PER_S
    pltpu.sync_copy(table_hbm.at[pl.ds(trow0, _T_PER_S)], buf_a)
    _normalize_rows(buf_a, _T_PER_S)
    pltpu.sync_copy(buf_a, ptable_hbm.at[pl.ds(trow0, _T_PER_S)])
    plsc.subcore_barrier()

    base_row = wid * _B_PER_W
    pltpu.sync_copy(idx_hbm.at[pl.ds(base_row, _B_PER_W)], idx_v)

    bufs = (buf_a, buf_b)
    osems = (osem_a, osem_b)
    out_cps = [None, None]
    for c in range(_N_CHUNKS):
        buf = bufs[c % 2]
        if out_cps[c % 2] is not None:
            out_cps[c % 2].wait()
        pltpu.async_copy(
            ptable_hbm.at[idx_v.at[pl.ds(c * _CH, _CH)]], buf, gsem
        ).wait()
        out_cps[c % 2] = pltpu.async_copy(
            buf, out_hbm.at[pl.ds(base_row + c * _CH, _CH)], osems[c % 2]
        )
    out_cps[0].wait()
    out_cps[1].wait()


@functools.partial(
    pl.kernel,
    mesh=plsc.VectorSubcoreMesh(core_axis_name="c", subcore_axis_name="s"),
    out_type=(
        jax.ShapeDtypeStruct((_BS, _D), jnp.float32),
        jax.ShapeDtypeStruct((_N_TASKS_PAD, _D), jnp.float32),
    ),
    scratch_types=[
        pltpu.VMEM((_B_PER_W,), jnp.int32),
        pltpu.VMEM((_CH, _D), jnp.float32),
        pltpu.VMEM((_CH, _D), jnp.float32),
        pltpu.SemaphoreType.DMA,
        pltpu.SemaphoreType.DMA,
        pltpu.SemaphoreType.DMA,
    ],
)
def _poly_selector(idx_hbm, table_hbm, out_hbm, ptable_hbm, idx_v, buf_a, buf_b,
                   gsem, osem_a, osem_b):
    _sc_body(idx_hbm, table_hbm, out_hbm, ptable_hbm, idx_v, buf_a, buf_b,
             gsem, osem_a, osem_b)


def kernel(routing_info, weights):
    idx = routing_info.reshape(-1).astype(jnp.int32)
    wpad = jnp.pad(weights, ((0, _N_TASKS_PAD - _N_TASKS), (0, 0)))
    out, _ = _poly_selector(idx, wpad)
    return out.reshape(_BS, _N_SPLITS, _N_SKILLS)
